# baseline (speedup 1.0000x reference)
import jax
import jax.numpy as jnp
from jax import lax
from jax.experimental import pallas as pl
from jax.experimental.pallas import tpu as pltpu
from jax.experimental.pallas import tpu_sc as plsc

K = 128
N = 8192
B = 64
THRESHOLD = 0.5

NC = 2
NS = 16
NW = NC * NS
ROWS_PER_W = B // NW
NCHUNK = N // 16
LIST_PAD = 144
CAP = 256
BK_PAD = CAP + 16


def _mask_body(x_ref, keys_ref):
    x = x_ref[:]
    probs = 0.5 + 0.5 * jnp.tanh(0.5 * x)
    mp = jnp.where(probs > THRESHOLD, probs, 0.0)
    keys_ref[:] = lax.bitcast_convert_type(mp, jnp.int32)


def _score_body(keys_ref, w_ref, k_ref, score_ref, prob_ref):
    keys = keys_ref[:]
    probs = lax.bitcast_convert_type(keys, jnp.float32)
    prob_ref[:] = probs
    slot = lax.broadcasted_iota(jnp.int32, (B, K), 1)
    valid = ((probs > THRESHOLD) & (slot < k_ref[0])).astype(jnp.float32)
    score_ref[:] = w_ref[0] * jnp.log(probs + 1e-9) * valid


def _sc_topk_body(keys_hbm, kw_hbm, okeys_hbm, oidx_hbm, oscore_hbm,
                  key_v, hist2_v, histm_v, ak_v, ai_v, bk_v, bi_v,
                  ok_v, oi_v, sm_v, kw_v, os_v):
    wid = lax.axis_index("s") * NC + lax.axis_index("c")
    lane = lax.iota(jnp.int32, 16)
    zeros16 = jnp.zeros((16,), jnp.int32)
    pltpu.sync_copy(kw_hbm, kw_v)
    kvec = kw_v[pl.ds(0, 16)].astype(jnp.int32)
    wvec = kw_v[pl.ds(16, 16)]

    def clear_hist(i):
        hist2_v[pl.ds(i * 16, 16)] = zeros16

    def merge_hist(c):
        acc = zeros16
        for ln in range(16):
            acc = acc + hist2_v[pl.ds(ln * 256 + c * 16, 16)]
        histm_v[pl.ds(c * 16, 16)] = acc

    def select_level(t):
        def body(i, carry):
            carry_sum, bsel, above = carry
            c = 15 - i
            v = histm_v[pl.ds(c * 16, 16)]
            tot = jnp.sum(v)
            csum = plsc.cumsum(v)
            sfx = carry_sum + (tot - csum) + v
            cand = jnp.max(jnp.where(sfx >= t, lane, -1))
            found = (bsel < 0) & (cand >= 0)
            above_new = carry_sum + jnp.sum(jnp.where(lane > cand, v, 0))
            bsel = jnp.where(found, c * 16 + cand, bsel)
            above = jnp.where(found, above_new, above)
            return (carry_sum + tot, bsel, above)

        _, bsel, above = lax.fori_loop(0, 16, body, (0, -1, 0))
        return bsel, above

    def hist_pass(level, bsel0, bsel1):
        plsc.parallel_loop(0, 256, unroll=8)(clear_hist)

        def body(i):
            kv = key_v[pl.ds(i * 16, 16)]
            b0 = jnp.where(kv == 0, 0, ((kv >> 16) & 0xFF) + 1)
            if level == 0:
                b, add = b0, jnp.full((16,), 1, jnp.int32)
            elif level == 1:
                b = (kv >> 8) & 0xFF
                add = jnp.where(b0 == bsel0, 1, 0)
            else:
                b = kv & 0xFF
                add = jnp.where((b0 == bsel0) & (((kv >> 8) & 0xFF) == bsel1),
                                1, 0)
            plsc.addupdate_scatter(hist2_v, [lane * 256 + b], add)

        plsc.parallel_loop(0, NCHUNK, unroll=8)(body)
        plsc.parallel_loop(0, 16, unroll=2)(merge_hist)

    def process_row(row, _):
        pltpu.sync_copy(keys_hbm.at[row], key_v)

        hist_pass(0, 0, 0)
        bsel0, above0 = select_level(128)
        t1 = 128 - above0

        def cp2(i, carry):
            pa, pe = carry
            kv = key_v[pl.ds(i * 16, 16)]
            idxv = i * 16 + lane
            b0v = jnp.where(kv == 0, 0, ((kv >> 16) & 0xFF) + 1)
            gt = b0v > bsel0
            inb = b0v == bsel0
            pos_gt = pa + plsc.cumsum(gt.astype(jnp.int32)) - 1
            pos_in = pe + plsc.cumsum(inb.astype(jnp.int32)) - 1
            inc = inb & (pos_in < CAP)
            plsc.store_scatter(ak_v, [pos_gt], kv, mask=gt)
            plsc.store_scatter(ai_v, [pos_gt], idxv, mask=gt)
            plsc.store_scatter(bk_v, [pos_in], kv, mask=inc)
            plsc.store_scatter(bi_v, [pos_in], idxv, mask=inc)
            return (pa + plsc.all_reduce_population_count(gt),
                    pe + plsc.all_reduce_population_count(inb))

        _, pe_fin = plsc.parallel_loop(0, NCHUNK, unroll=4,
                                       carry=(zeros16, zeros16))(
            lambda i, c: cp2(i, c))
        m_cnt = jnp.max(pe_fin)

        sm_v[0] = above0
        sm_v[1] = 0

        @pl.when((bsel0 > 0) & (m_cnt > CAP))
        def _():
            hist_pass(1, bsel0, 0)
            bsel1, above1 = select_level(t1)
            hist_pass(2, bsel0, bsel1)
            bsel2, above2 = select_level(t1 - above1)
            cutkey = 0x3F000000 | ((bsel0 - 1) << 16) | (bsel1 << 8) | bsel2

            def cp3(i, carry):
                pa, pe = carry
                kv = key_v[pl.ds(i * 16, 16)]
                idxv = i * 16 + lane
                b0v = jnp.where(kv == 0, 0, ((kv >> 16) & 0xFF) + 1)
                gtc = (b0v == bsel0) & (kv > cutkey)
                eqc = kv == cutkey
                pos_g = pa + plsc.cumsum(gtc.astype(jnp.int32)) - 1
                pos_e = pe + plsc.cumsum(eqc.astype(jnp.int32)) - 1
                plsc.store_scatter(ak_v, [pos_g], kv, mask=gtc)
                plsc.store_scatter(ai_v, [pos_g], idxv, mask=gtc)
                plsc.store_scatter(bi_v, [pos_e], idxv,
                                   mask=eqc & (pos_e < CAP))
                return (pa + plsc.all_reduce_population_count(gtc),
                        pe + plsc.all_reduce_population_count(eqc))

            plsc.parallel_loop(0, NCHUNK, unroll=4,
                               carry=(zeros16 + above0, zeros16))(
                lambda i, c: cp3(i, c))
            sm_v[0] = above0 + above1 + above2
            sm_v[1] = cutkey

        a_star = sm_v[0]
        fill_key = sm_v[1]

        nlc = LIST_PAD // 16
        kcis = [ak_v[pl.ds(lc * 16, 16)] for lc in range(nlc)]
        poscis = [lc * 16 + lane for lc in range(nlc)]

        def rk(j, ranks):
            kj = plsc.load_gather(ak_v, [zeros16 + j])
            out = []
            for lc in range(nlc):
                beat = (kj > kcis[lc]) | ((kj == kcis[lc]) & (j < poscis[lc]))
                out.append(ranks[lc] + beat.astype(jnp.int32))
            return tuple(out)

        ranks = plsc.parallel_loop(0, a_star, unroll=2,
                                   carry=(zeros16,) * nlc)(rk)
        for lc in range(nlc):
            vmask = poscis[lc] < a_star
            plsc.store_scatter(ok_v, [ranks[lc]], kcis[lc], mask=vmask)
            plsc.store_scatter(oi_v, [ranks[lc]],
                               ai_v[pl.ds(lc * 16, 16)], mask=vmask)

        @pl.when((bsel0 > 0) & (m_cnt <= CAP))
        def _():
            def outer(lc, _):
                kci = bk_v[pl.ds(lc * 16, 16)]
                posci = lc * 16 + lane

                def rk2(j, rank):
                    kj = plsc.load_gather(bk_v, [zeros16 + j])
                    beat = (kj > kci) | ((kj == kci) & (j < posci))
                    return rank + beat.astype(jnp.int32)

                rank = lax.fori_loop(0, m_cnt, rk2, zeros16)
                keep = (posci < m_cnt) & (rank < t1)
                plsc.store_scatter(ok_v, [above0 + rank], kci, mask=keep)
                plsc.store_scatter(oi_v, [above0 + rank],
                                   bi_v[pl.ds(lc * 16, 16)], mask=keep)
                return 0

            lax.fori_loop(0, (m_cnt + 15) // 16, outer, 0)

        @pl.when((bsel0 == 0) | (m_cnt > CAP))
        def _():
            for oc in range(K // 16):
                slot = oc * 16 + lane
                need = slot >= a_star
                eqpos = jnp.where(need, slot - a_star, 0)
                gi = plsc.load_gather(bi_v, [eqpos], mask=need)
                cur_k = ok_v[pl.ds(oc * 16, 16)]
                cur_i = oi_v[pl.ds(oc * 16, 16)]
                ok_v[pl.ds(oc * 16, 16)] = jnp.where(need, fill_key, cur_k)
                oi_v[pl.ds(oc * 16, 16)] = jnp.where(need, gi, cur_i)

        for oc in range(K // 16):
            p = lax.bitcast_convert_type(ok_v[pl.ds(oc * 16, 16)],
                                         jnp.float32)
            s = (p - 1.0) / (p + 1.0)
            s2 = s * s
            ln = 2.0 * s * (1.0 + s2 * (1.0 / 3.0 + s2 * (
                1.0 / 5.0 + s2 * (1.0 / 7.0 + s2 * (1.0 / 9.0)))))
            slot = oc * 16 + lane
            valid = (p > THRESHOLD) & (slot < kvec)
            os_v[pl.ds(oc * 16, 16)] = jnp.where(valid, wvec * ln, 0.0)

        pltpu.sync_copy(ok_v, okeys_hbm.at[row])
        pltpu.sync_copy(oi_v, oidx_hbm.at[row])
        pltpu.sync_copy(os_v, oscore_hbm.at[row])
        return 0

    lax.fori_loop(wid * ROWS_PER_W, (wid + 1) * ROWS_PER_W, process_row, 0)


def _sc_topk(keys, kw):
    mesh = plsc.VectorSubcoreMesh(core_axis_name="c", subcore_axis_name="s",
                                  num_cores=NC, num_subcores=NS)
    f = pl.kernel(
        _sc_topk_body,
        out_type=(
            jax.ShapeDtypeStruct((B, K), jnp.int32),
            jax.ShapeDtypeStruct((B, K), jnp.int32),
            jax.ShapeDtypeStruct((B, K), jnp.float32),
        ),
        mesh=mesh,
        compiler_params=pltpu.CompilerParams(needs_layout_passes=False),
        scratch_types=[
            pltpu.VMEM((N,), jnp.int32),
            pltpu.VMEM((4096,), jnp.int32),
            pltpu.VMEM((256,), jnp.int32),
            pltpu.VMEM((LIST_PAD,), jnp.int32),
            pltpu.VMEM((LIST_PAD,), jnp.int32),
            pltpu.VMEM((BK_PAD,), jnp.int32),
            pltpu.VMEM((BK_PAD,), jnp.int32),
            pltpu.VMEM((K,), jnp.int32),
            pltpu.VMEM((K,), jnp.int32),
            pltpu.SMEM((2,), jnp.int32),
            pltpu.VMEM((32,), jnp.float32),
            pltpu.VMEM((K,), jnp.float32),
        ],
    )
    return f(keys, kw)


def kernel(logits, w_spl, k):
    keys = pl.pallas_call(
        _mask_body,
        out_shape=jax.ShapeDtypeStruct((B, N), jnp.int32),
        in_specs=[pl.BlockSpec(memory_space=pltpu.VMEM)],
        out_specs=pl.BlockSpec(memory_space=pltpu.VMEM),
    )(logits)

    kw = jnp.concatenate([
        jnp.full((16,), k, jnp.float32),
        jnp.full((16,), w_spl, jnp.float32),
    ])
    okeys, oidx, oscores = _sc_topk(keys, kw)
    probs = lax.bitcast_convert_type(okeys, jnp.float32)
    return oscores, probs, oidx


# speedup vs baseline: 1.0305x; 1.0305x over previous
import jax
import jax.numpy as jnp
from jax import lax
from jax.experimental import pallas as pl
from jax.experimental.pallas import tpu as pltpu
from jax.experimental.pallas import tpu_sc as plsc

K = 128
N = 8192
B = 64
THRESHOLD = 0.5

NC = 2
NS = 16
NW = NC * NS
ROWS_PER_W = B // NW
NCHUNK = N // 16
LIST_PAD = 144
CAP = 256
BK_PAD = CAP + 16


def _mask_body(x_ref, keys_ref):
    x = x_ref[:]
    probs = 0.5 + 0.5 * jnp.tanh(0.5 * x)
    mp = jnp.where(probs > THRESHOLD, probs, 0.0)
    keys_ref[:] = lax.bitcast_convert_type(mp, jnp.int32)


def _score_body(keys_ref, w_ref, k_ref, score_ref, prob_ref):
    keys = keys_ref[:]
    probs = lax.bitcast_convert_type(keys, jnp.float32)
    prob_ref[:] = probs
    slot = lax.broadcasted_iota(jnp.int32, (B, K), 1)
    valid = ((probs > THRESHOLD) & (slot < k_ref[0])).astype(jnp.float32)
    score_ref[:] = w_ref[0] * jnp.log(probs + 1e-9) * valid


def _sc_topk_body(keys_hbm, okeys_hbm, oidx_hbm,
                  key_v, hist2_v, histm_v, ak_v, ai_v, bk_v, bi_v,
                  ok_v, oi_v, sm_v):
    wid = lax.axis_index("s") * NC + lax.axis_index("c")
    lane = lax.iota(jnp.int32, 16)
    zeros16 = jnp.zeros((16,), jnp.int32)

    def clear_hist(i):
        hist2_v[pl.ds(i * 16, 16)] = zeros16

    def merge_hist(c):
        acc = zeros16
        for ln in range(16):
            acc = acc + hist2_v[pl.ds(ln * 256 + c * 16, 16)]
        histm_v[pl.ds(c * 16, 16)] = acc

    def select_level(t):
        def body(i, carry):
            carry_sum, bsel, above = carry
            c = 15 - i
            v = histm_v[pl.ds(c * 16, 16)]
            tot = jnp.sum(v)
            csum = plsc.cumsum(v)
            sfx = carry_sum + (tot - csum) + v
            cand = jnp.max(jnp.where(sfx >= t, lane, -1))
            found = (bsel < 0) & (cand >= 0)
            above_new = carry_sum + jnp.sum(jnp.where(lane > cand, v, 0))
            bsel = jnp.where(found, c * 16 + cand, bsel)
            above = jnp.where(found, above_new, above)
            return (carry_sum + tot, bsel, above)

        _, bsel, above = lax.fori_loop(0, 16, body, (0, -1, 0))
        return bsel, above

    def hist_pass(level, bsel0, bsel1):
        plsc.parallel_loop(0, 256, unroll=8)(clear_hist)

        def body(i):
            kv = key_v[pl.ds(i * 16, 16)]
            b0 = jnp.where(kv == 0, 0, ((kv >> 16) & 0xFF) + 1)
            if level == 0:
                b, add = b0, jnp.full((16,), 1, jnp.int32)
            elif level == 1:
                b = (kv >> 8) & 0xFF
                add = jnp.where(b0 == bsel0, 1, 0)
            else:
                b = kv & 0xFF
                add = jnp.where((b0 == bsel0) & (((kv >> 8) & 0xFF) == bsel1),
                                1, 0)
            plsc.addupdate_scatter(hist2_v, [lane * 256 + b], add)

        plsc.parallel_loop(0, NCHUNK, unroll=8)(body)
        plsc.parallel_loop(0, 16, unroll=2)(merge_hist)

    def process_row(row, _):
        pltpu.sync_copy(keys_hbm.at[row], key_v)

        hist_pass(0, 0, 0)
        bsel0, above0 = select_level(128)
        t1 = 128 - above0

        def cp2(i, carry):
            pa, pe = carry
            kv = key_v[pl.ds(i * 16, 16)]
            idxv = i * 16 + lane
            b0v = jnp.where(kv == 0, 0, ((kv >> 16) & 0xFF) + 1)
            gt = b0v > bsel0
            inb = b0v == bsel0
            pos_gt = pa + plsc.cumsum(gt.astype(jnp.int32)) - 1
            pos_in = pe + plsc.cumsum(inb.astype(jnp.int32)) - 1
            inc = inb & (pos_in < CAP)
            plsc.store_scatter(ak_v, [pos_gt], kv, mask=gt)
            plsc.store_scatter(ai_v, [pos_gt], idxv, mask=gt)
            plsc.store_scatter(bk_v, [pos_in], kv, mask=inc)
            plsc.store_scatter(bi_v, [pos_in], idxv, mask=inc)
            return (pa + plsc.all_reduce_population_count(gt),
                    pe + plsc.all_reduce_population_count(inb))

        _, pe_fin = plsc.parallel_loop(0, NCHUNK, unroll=4,
                                       carry=(zeros16, zeros16))(
            lambda i, c: cp2(i, c))
        m_cnt = jnp.max(pe_fin)

        sm_v[0] = above0
        sm_v[1] = 0

        @pl.when((bsel0 > 0) & (m_cnt > CAP))
        def _():
            hist_pass(1, bsel0, 0)
            bsel1, above1 = select_level(t1)
            hist_pass(2, bsel0, bsel1)
            bsel2, above2 = select_level(t1 - above1)
            cutkey = 0x3F000000 | ((bsel0 - 1) << 16) | (bsel1 << 8) | bsel2

            def cp3(i, carry):
                pa, pe = carry
                kv = key_v[pl.ds(i * 16, 16)]
                idxv = i * 16 + lane
                b0v = jnp.where(kv == 0, 0, ((kv >> 16) & 0xFF) + 1)
                gtc = (b0v == bsel0) & (kv > cutkey)
                eqc = kv == cutkey
                pos_g = pa + plsc.cumsum(gtc.astype(jnp.int32)) - 1
                pos_e = pe + plsc.cumsum(eqc.astype(jnp.int32)) - 1
                plsc.store_scatter(ak_v, [pos_g], kv, mask=gtc)
                plsc.store_scatter(ai_v, [pos_g], idxv, mask=gtc)
                plsc.store_scatter(bi_v, [pos_e], idxv,
                                   mask=eqc & (pos_e < CAP))
                return (pa + plsc.all_reduce_population_count(gtc),
                        pe + plsc.all_reduce_population_count(eqc))

            plsc.parallel_loop(0, NCHUNK, unroll=4,
                               carry=(zeros16 + above0, zeros16))(
                lambda i, c: cp3(i, c))
            sm_v[0] = above0 + above1 + above2
            sm_v[1] = cutkey

        a_star = sm_v[0]
        fill_key = sm_v[1]

        nlc = LIST_PAD // 16
        kcis = [ak_v[pl.ds(lc * 16, 16)] for lc in range(nlc)]
        poscis = [lc * 16 + lane for lc in range(nlc)]

        def rk(j, ranks):
            kj = plsc.load_gather(ak_v, [zeros16 + j])
            out = []
            for lc in range(nlc):
                beat = (kj > kcis[lc]) | ((kj == kcis[lc]) & (j < poscis[lc]))
                out.append(ranks[lc] + beat.astype(jnp.int32))
            return tuple(out)

        ranks = plsc.parallel_loop(0, a_star, unroll=2,
                                   carry=(zeros16,) * nlc)(rk)
        for lc in range(nlc):
            vmask = poscis[lc] < a_star
            plsc.store_scatter(ok_v, [ranks[lc]], kcis[lc], mask=vmask)
            plsc.store_scatter(oi_v, [ranks[lc]],
                               ai_v[pl.ds(lc * 16, 16)], mask=vmask)

        @pl.when((bsel0 > 0) & (m_cnt <= CAP))
        def _():
            def outer(lc, _):
                kci = bk_v[pl.ds(lc * 16, 16)]
                posci = lc * 16 + lane

                def rk2(j, rank):
                    kj = plsc.load_gather(bk_v, [zeros16 + j])
                    beat = (kj > kci) | ((kj == kci) & (j < posci))
                    return rank + beat.astype(jnp.int32)

                rank = lax.fori_loop(0, m_cnt, rk2, zeros16)
                keep = (posci < m_cnt) & (rank < t1)
                plsc.store_scatter(ok_v, [above0 + rank], kci, mask=keep)
                plsc.store_scatter(oi_v, [above0 + rank],
                                   bi_v[pl.ds(lc * 16, 16)], mask=keep)
                return 0

            lax.fori_loop(0, (m_cnt + 15) // 16, outer, 0)

        @pl.when((bsel0 == 0) | (m_cnt > CAP))
        def _():
            for oc in range(K // 16):
                slot = oc * 16 + lane
                need = slot >= a_star
                eqpos = jnp.where(need, slot - a_star, 0)
                gi = plsc.load_gather(bi_v, [eqpos], mask=need)
                cur_k = ok_v[pl.ds(oc * 16, 16)]
                cur_i = oi_v[pl.ds(oc * 16, 16)]
                ok_v[pl.ds(oc * 16, 16)] = jnp.where(need, fill_key, cur_k)
                oi_v[pl.ds(oc * 16, 16)] = jnp.where(need, gi, cur_i)

        pltpu.sync_copy(ok_v, okeys_hbm.at[row])
        pltpu.sync_copy(oi_v, oidx_hbm.at[row])
        return 0

    lax.fori_loop(wid * ROWS_PER_W, (wid + 1) * ROWS_PER_W, process_row, 0)


def _sc_topk(keys):
    mesh = plsc.VectorSubcoreMesh(core_axis_name="c", subcore_axis_name="s",
                                  num_cores=NC, num_subcores=NS)
    f = pl.kernel(
        _sc_topk_body,
        out_type=(
            jax.ShapeDtypeStruct((B, K), jnp.int32),
            jax.ShapeDtypeStruct((B, K), jnp.int32),
        ),
        mesh=mesh,
        compiler_params=pltpu.CompilerParams(needs_layout_passes=False),
        scratch_types=[
            pltpu.VMEM((N,), jnp.int32),
            pltpu.VMEM((4096,), jnp.int32),
            pltpu.VMEM((256,), jnp.int32),
            pltpu.VMEM((LIST_PAD,), jnp.int32),
            pltpu.VMEM((LIST_PAD,), jnp.int32),
            pltpu.VMEM((BK_PAD,), jnp.int32),
            pltpu.VMEM((BK_PAD,), jnp.int32),
            pltpu.VMEM((K,), jnp.int32),
            pltpu.VMEM((K,), jnp.int32),
            pltpu.SMEM((2,), jnp.int32),
        ],
    )
    return f(keys)


def kernel(logits, w_spl, k):
    keys = pl.pallas_call(
        _mask_body,
        out_shape=jax.ShapeDtypeStruct((B, N), jnp.int32),
        in_specs=[pl.BlockSpec(memory_space=pltpu.VMEM)],
        out_specs=pl.BlockSpec(memory_space=pltpu.VMEM),
    )(logits)

    okeys, oidx = _sc_topk(keys)

    w = jnp.asarray(w_spl, jnp.float32).reshape(1)
    kk = jnp.asarray(k, jnp.int32).reshape(1)
    scores, probs = pl.pallas_call(
        _score_body,
        out_shape=(
            jax.ShapeDtypeStruct((B, K), jnp.float32),
            jax.ShapeDtypeStruct((B, K), jnp.float32),
        ),
        in_specs=[
            pl.BlockSpec(memory_space=pltpu.VMEM),
            pl.BlockSpec(memory_space=pltpu.SMEM),
            pl.BlockSpec(memory_space=pltpu.SMEM),
        ],
        out_specs=(
            pl.BlockSpec(memory_space=pltpu.VMEM),
            pl.BlockSpec(memory_space=pltpu.VMEM),
        ),
    )(okeys, w, kk)
    return scores, probs, oidx
